# baseline (speedup 1.0000x reference)
import functools

import jax
import jax.numpy as jnp
from jax import lax
from jax.experimental import pallas as pl
from jax.experimental.pallas import tpu as pltpu
from jax.experimental.pallas import tpu_sc as plsc

NC = 2
NS = 16
NW = NC * NS
L = 16
K = 80


def _sc_spmm(n_pad, d_feat, n_chunks):
    rows_per_tile = n_pad // NS
    assert n_pad % (NS * 8) == 0 and d_feat % L == 0 and n_chunks % 4 == 0
    assert n_chunks >= 8
    groups = d_feat // L
    mesh = plsc.VectorSubcoreMesh(
        core_axis_name="c", subcore_axis_name="s", num_cores=NC,
        num_subcores=NS)

    @functools.partial(
        pl.kernel,
        out_type=jax.ShapeDtypeStruct((NC, n_pad, d_feat), jnp.float32),
        mesh=mesh,
        scratch_types=dict(
            c0=pltpu.VMEM((K,), jnp.int32),
            c1=pltpu.VMEM((K,), jnp.int32),
            c2=pltpu.VMEM((K,), jnp.int32),
            c3=pltpu.VMEM((K,), jnp.int32),
            r0=pltpu.VMEM((K,), jnp.int32),
            r1=pltpu.VMEM((K,), jnp.int32),
            r2=pltpu.VMEM((K,), jnp.int32),
            r3=pltpu.VMEM((K,), jnp.int32),
            v0=pltpu.VMEM((K,), jnp.float32),
            v1=pltpu.VMEM((K,), jnp.float32),
            v2=pltpu.VMEM((K,), jnp.float32),
            v3=pltpu.VMEM((K,), jnp.float32),
            ra=pltpu.VMEM((K,), jnp.int32),
            rb=pltpu.VMEM((K,), jnp.int32),
            ga=pltpu.VMEM((K, d_feat), jnp.float32),
            gb=pltpu.VMEM((K, d_feat), jnp.float32),
            sa=pltpu.VMEM((K, d_feat), jnp.float32),
            sb=pltpu.VMEM((K, d_feat), jnp.float32),
            acc=pltpu.VMEM_SHARED((n_pad, d_feat), jnp.float32),
            gsem_a=pltpu.SemaphoreType.DMA,
            gsem_b=pltpu.SemaphoreType.DMA,
            ssem_a=pltpu.SemaphoreType.DMA,
            ssem_b=pltpu.SemaphoreType.DMA,
            esem0=pltpu.SemaphoreType.DMA,
            esem1=pltpu.SemaphoreType.DMA,
            esem2=pltpu.SemaphoreType.DMA,
            esem3=pltpu.SemaphoreType.DMA,
        ),
    )
    def spmm(embeds_hbm, col_hbm, row_hbm, vals_hbm, out_hbm, *, c0, c1,
             c2, c3, r0, r1, r2, r3, v0, v1, v2, v3, ra, rb, ga, gb, sa, sb,
             acc, gsem_a, gsem_b, ssem_a, ssem_b, esem0, esem1, esem2,
             esem3):
        cid = lax.axis_index("c")
        sid = lax.axis_index("s")
        wid = sid * NC + cid
        cbufs = (c0, c1, c2, c3)
        rebufs = (r0, r1, r2, r3)
        vbufs = (v0, v1, v2, v3)
        esems = (esem0, esem1, esem2, esem3)
        gbufs = (ga, gb)
        gsems = (gsem_a, gsem_b)
        sbufs = (sa, sb)
        ssems = (ssem_a, ssem_b)
        rbufs = (ra, rb)

        def load_edges(j, t):
            pltpu.async_copy(col_hbm.at[wid, j], cbufs[t], esems[t])
            pltpu.async_copy(row_hbm.at[wid, j], rebufs[t], esems[t])
            pltpu.async_copy(vals_hbm.at[wid, j], vbufs[t], esems[t])

        def wait_edges(t):
            pltpu.make_async_copy(
                col_hbm.at[0, 0], cbufs[t], esems[t]).wait()
            pltpu.make_async_copy(
                row_hbm.at[0, 0], rebufs[t], esems[t]).wait()
            pltpu.make_async_copy(
                vals_hbm.at[0, 0], vbufs[t], esems[t]).wait()

        def issue_gather(t, b):
            pltpu.async_copy(embeds_hbm.at[cbufs[t]], gbufs[b], gsems[b])

        def wait_gather(b):
            pltpu.make_async_copy(embeds_hbm.at[c0], gbufs[b],
                                  gsems[b]).wait()

        def issue_scatter(b):
            pltpu.make_async_copy(sbufs[b], acc.at[rbufs[b]],
                                  ssems[b]).start(add=True)

        def wait_scatter(b):
            pltpu.make_async_copy(sa, acc.at[ra], ssems[b]).wait()

        def scale(t, b):
            src, dst, vv = gbufs[b], sbufs[b], vbufs[t]

            for q in range(K // L):
                v16 = vv[pl.ds(q * L, L)]
                for tt in range(L):
                    i = q * L + tt
                    bv = jnp.full((L,), v16[tt], jnp.float32)
                    for g in range(groups):
                        sl = pl.ds(g * L, L)
                        dst[i, sl] = src[i, sl] * bv

        def copy_rows(t, b):
            for g in range(K // L):
                sl = pl.ds(g * L, L)
                rbufs[b][sl] = rebufs[t][sl]

        for t in range(4):
            load_edges(t, t)

        zv = jnp.zeros((L,), jnp.float32)

        def zero_row(i, _):
            for g in range(groups):
                sa[i, pl.ds(g * L, L)] = zv
            return 0

        lax.fori_loop(0, K, zero_row, 0)
        zrows = next(z for z in range(min(K, rows_per_tile), 0, -1)
                     if rows_per_tile % z == 0)
        for c in range(rows_per_tile // zrows):
            pltpu.sync_copy(
                sa.at[pl.ds(0, zrows)],
                acc.at[pl.ds(sid * rows_per_tile + c * zrows, zrows)])
        plsc.subcore_barrier()

        wait_edges(0)
        issue_gather(0, 0)
        wait_edges(1)
        issue_gather(1, 1)

        def step(j, t, b):
            with jax.named_scope("wait_gather"):
                wait_gather(b)

            @pl.when(j >= 2)
            def _():
                with jax.named_scope("wait_scatter"):
                    wait_scatter(b)

            with jax.named_scope("scale"):
                scale(t, b)
            copy_rows(t, b)

            @pl.when(j + 4 < n_chunks)
            def _():
                load_edges(j + 4, t)

            @pl.when(j + 2 < n_chunks)
            def _():
                wait_edges((t + 2) % 4)
                issue_gather((t + 2) % 4, b)

            issue_scatter(b)

        def quad(qq, _):
            j = 4 * qq
            for tt in range(4):
                step(j + tt, tt, tt % 2)
            return 0

        lax.fori_loop(0, n_chunks // 4, quad, 0)
        wait_scatter(0)
        wait_scatter(1)

        plsc.subcore_barrier()
        base = sid * rows_per_tile
        pltpu.sync_copy(acc.at[pl.ds(base, rows_per_tile)],
                        out_hbm.at[cid].at[pl.ds(base, rows_per_tile)])

    return spmm


def _combine(partials, n_nodes):
    d_feat = partials.shape[2]
    blk = 1000
    assert n_nodes % blk == 0

    def body(a_ref, b_ref, o_ref):
        o_ref[...] = a_ref[...] + b_ref[...]

    return pl.pallas_call(
        body,
        grid=(n_nodes // blk,),
        in_specs=[pl.BlockSpec((blk, d_feat), lambda i: (i, 0))] * 2,
        out_specs=pl.BlockSpec((blk, d_feat), lambda i: (i, 0)),
        out_shape=jax.ShapeDtypeStruct((n_nodes, d_feat), jnp.float32),
    )(partials[0], partials[1])


def kernel(edge_index, edge_vals, embeds):
    n_edges = edge_vals.shape[0]
    n_nodes, d_feat = embeds.shape

    per_super = NW * K * 4
    n_super = -(-n_edges // per_super)
    e_pad = n_super * per_super
    n_chunks = e_pad // (NW * K)
    pad = e_pad - n_edges

    row = edge_index[0]
    col = edge_index[1]
    vals = edge_vals
    if pad:
        spread = (jnp.arange(pad, dtype=jnp.int32) * 16) % n_nodes
        row = jnp.concatenate([row, spread])
        col = jnp.concatenate([col, spread])
        vals = jnp.concatenate([vals, jnp.zeros((pad,), vals.dtype)])
    col = col.reshape(NW, n_chunks, K)
    row = row.reshape(NW, n_chunks, K)
    vals = vals.reshape(NW, n_chunks, K)

    n_pad = -(-n_nodes // (NS * K)) * (NS * K)
    partials = _sc_spmm(n_pad, d_feat, n_chunks)(embeds, col, row, vals)
    return _combine(partials, n_nodes)


# speedup vs baseline: 1.0536x; 1.0536x over previous
import functools

import jax
import jax.numpy as jnp
from jax import lax
from jax.experimental import pallas as pl
from jax.experimental.pallas import tpu as pltpu
from jax.experimental.pallas import tpu_sc as plsc

NC = 2
NS = 16
NW = NC * NS
L = 16
K = 80


def _sc_spmm(n_pad, d_feat, n_chunks):
    rows_per_tile = n_pad // NS
    assert n_pad % (NS * 8) == 0 and d_feat % L == 0
    assert n_chunks >= 8
    groups = d_feat // L
    mesh = plsc.VectorSubcoreMesh(
        core_axis_name="c", subcore_axis_name="s", num_cores=NC,
        num_subcores=NS)

    @functools.partial(
        pl.kernel,
        out_type=(jax.ShapeDtypeStruct((n_pad, d_feat), jnp.float32),
                  jax.ShapeDtypeStruct((n_pad, d_feat), jnp.float32)),
        mesh=mesh,
        scratch_types=dict(
            c0=pltpu.VMEM((K,), jnp.int32),
            c1=pltpu.VMEM((K,), jnp.int32),
            c2=pltpu.VMEM((K,), jnp.int32),
            c3=pltpu.VMEM((K,), jnp.int32),
            r0=pltpu.VMEM((K,), jnp.int32),
            r1=pltpu.VMEM((K,), jnp.int32),
            r2=pltpu.VMEM((K,), jnp.int32),
            r3=pltpu.VMEM((K,), jnp.int32),
            v0=pltpu.VMEM((K,), jnp.float32),
            v1=pltpu.VMEM((K,), jnp.float32),
            v2=pltpu.VMEM((K,), jnp.float32),
            v3=pltpu.VMEM((K,), jnp.float32),
            ra=pltpu.VMEM((K,), jnp.int32),
            rb=pltpu.VMEM((K,), jnp.int32),
            ga=pltpu.VMEM((K, d_feat), jnp.float32),
            gb=pltpu.VMEM((K, d_feat), jnp.float32),
            sa=pltpu.VMEM((K, d_feat), jnp.float32),
            sb=pltpu.VMEM((K, d_feat), jnp.float32),
            acc=pltpu.VMEM_SHARED((n_pad, d_feat), jnp.float32),
            gsem_a=pltpu.SemaphoreType.DMA,
            gsem_b=pltpu.SemaphoreType.DMA,
            ssem_a=pltpu.SemaphoreType.DMA,
            ssem_b=pltpu.SemaphoreType.DMA,
            esem0=pltpu.SemaphoreType.DMA,
            esem1=pltpu.SemaphoreType.DMA,
            esem2=pltpu.SemaphoreType.DMA,
            esem3=pltpu.SemaphoreType.DMA,
        ),
    )
    def spmm(embeds_hbm, col_hbm, row_hbm, vals_hbm, out0_hbm, out1_hbm,
             *, c0, c1,
             c2, c3, r0, r1, r2, r3, v0, v1, v2, v3, ra, rb, ga, gb, sa, sb,
             acc, gsem_a, gsem_b, ssem_a, ssem_b, esem0, esem1, esem2,
             esem3):
        cid = lax.axis_index("c")
        sid = lax.axis_index("s")
        wid = sid * NC + cid
        cbufs = (c0, c1, c2, c3)
        rebufs = (r0, r1, r2, r3)
        vbufs = (v0, v1, v2, v3)
        esems = (esem0, esem1, esem2, esem3)
        gbufs = (ga, gb)
        gsems = (gsem_a, gsem_b)
        sbufs = (sa, sb)
        ssems = (ssem_a, ssem_b)
        rbufs = (ra, rb)

        def load_edges(j, t):
            pltpu.async_copy(col_hbm.at[wid, j], cbufs[t], esems[t])
            pltpu.async_copy(row_hbm.at[wid, j], rebufs[t], esems[t])
            pltpu.async_copy(vals_hbm.at[wid, j], vbufs[t], esems[t])

        def wait_edges(t):
            pltpu.make_async_copy(
                col_hbm.at[0, 0], cbufs[t], esems[t]).wait()
            pltpu.make_async_copy(
                row_hbm.at[0, 0], rebufs[t], esems[t]).wait()
            pltpu.make_async_copy(
                vals_hbm.at[0, 0], vbufs[t], esems[t]).wait()

        def issue_gather(t, b):
            pltpu.async_copy(embeds_hbm.at[cbufs[t]], gbufs[b], gsems[b])

        def wait_gather(b):
            pltpu.make_async_copy(embeds_hbm.at[c0], gbufs[b],
                                  gsems[b]).wait()

        def issue_scatter(b):
            pltpu.make_async_copy(sbufs[b], acc.at[rbufs[b]],
                                  ssems[b]).start(add=True)

        def wait_scatter(b):
            pltpu.make_async_copy(sa, acc.at[ra], ssems[b]).wait()

        def scale(t, b):
            src, dst, vv = gbufs[b], sbufs[b], vbufs[t]

            for q in range(K // L):
                v16 = vv[pl.ds(q * L, L)]
                for tt in range(L):
                    i = q * L + tt
                    bv = jnp.full((L,), v16[tt], jnp.float32)
                    for g in range(groups):
                        sl = pl.ds(g * L, L)
                        dst[i, sl] = src[i, sl] * bv

        def copy_rows(t, b):
            for g in range(K // L):
                sl = pl.ds(g * L, L)
                rbufs[b][sl] = rebufs[t][sl]

        for t in range(4):
            load_edges(t, t)

        zv = jnp.zeros((L,), jnp.float32)

        def zero_row(i, _):
            for g in range(groups):
                sa[i, pl.ds(g * L, L)] = zv
            return 0

        lax.fori_loop(0, K, zero_row, 0)
        zrows = next(z for z in range(min(K, rows_per_tile), 0, -1)
                     if rows_per_tile % z == 0)
        for c in range(rows_per_tile // zrows):
            pltpu.sync_copy(
                sa.at[pl.ds(0, zrows)],
                acc.at[pl.ds(sid * rows_per_tile + c * zrows, zrows)])
        plsc.subcore_barrier()

        wait_edges(0)
        issue_gather(0, 0)
        wait_edges(1)
        issue_gather(1, 1)

        def step(j, t, b):
            wait_gather(b)

            @pl.when(j >= 2)
            def _():
                wait_scatter(b)

            scale(t, b)
            copy_rows(t, b)

            @pl.when(j + 4 < n_chunks)
            def _():
                load_edges(j + 4, t)

            @pl.when(j + 2 < n_chunks)
            def _():
                wait_edges((t + 2) % 4)
                issue_gather((t + 2) % 4, b)

            issue_scatter(b)

        def quad(qq, _):
            j0 = 4 * qq
            for tt in range(4):
                j = j0 + tt

                @pl.when(j < n_chunks)
                def _():
                    step(j, tt, tt % 2)
            return 0

        lax.fori_loop(0, (n_chunks + 3) // 4, quad, 0)
        wait_scatter(0)
        wait_scatter(1)

        plsc.subcore_barrier()
        base = sid * rows_per_tile

        @pl.when(cid == 0)
        def _():
            pltpu.sync_copy(acc.at[pl.ds(base, rows_per_tile)],
                            out0_hbm.at[pl.ds(base, rows_per_tile)])

        @pl.when(cid == 1)
        def _():
            pltpu.sync_copy(acc.at[pl.ds(base, rows_per_tile)],
                            out1_hbm.at[pl.ds(base, rows_per_tile)])

    return spmm


def _combine(p0, p1, n_nodes):
    d_feat = p0.shape[1]
    blk = 1000
    assert n_nodes % blk == 0

    def body(a_ref, b_ref, o_ref):
        o_ref[...] = a_ref[...] + b_ref[...]

    return pl.pallas_call(
        body,
        grid=(n_nodes // blk,),
        in_specs=[pl.BlockSpec((blk, d_feat), lambda i: (i, 0))] * 2,
        out_specs=pl.BlockSpec((blk, d_feat), lambda i: (i, 0)),
        out_shape=jax.ShapeDtypeStruct((n_nodes, d_feat), jnp.float32),
    )(p0, p1)


def kernel(edge_index, edge_vals, embeds):
    n_edges = edge_vals.shape[0]
    n_nodes, d_feat = embeds.shape

    per_super = NW * K
    n_super = -(-n_edges // per_super)
    e_pad = n_super * per_super
    n_chunks = e_pad // (NW * K)
    pad = e_pad - n_edges

    row = edge_index[0]
    col = edge_index[1]
    vals = edge_vals
    if pad:
        spread = (jnp.arange(pad, dtype=jnp.int32) * 16) % n_nodes
        row = jnp.concatenate([row, spread])
        col = jnp.concatenate([col, spread])
        vals = jnp.concatenate([vals, jnp.zeros((pad,), vals.dtype)])
    col = col.reshape(NW, n_chunks, K)
    row = row.reshape(NW, n_chunks, K)
    vals = vals.reshape(NW, n_chunks, K)

    n_pad = -(-n_nodes // (NS * K)) * (NS * K)
    p0, p1 = _sc_spmm(n_pad, d_feat, n_chunks)(embeds, col, row, vals)
    return _combine(p0, p1, n_nodes)
